# baseline (speedup 1.0000x reference)
import jax
import jax.numpy as jnp
from jax.experimental import pallas as pl
from jax.experimental.pallas import tpu as pltpu

SEQ_LEN = 8192
D_MODEL = 768
BATCH = 4
EPS = 1e-12

BLOCK_ROWS = 1024


def _ln_kernel(x_ref, pos_ref, gamma_ref, beta_ref, out_ref):
    i = pl.program_id(1)
    x = x_ref[0] + pos_ref[pl.ds(i * BLOCK_ROWS, BLOCK_ROWS), :]
    inv_d = 1.0 / D_MODEL
    m = jnp.sum(x, axis=-1, keepdims=True) * inv_d
    m2 = jnp.sum(x * x, axis=-1, keepdims=True) * inv_d
    var = m2 - m * m
    rs = jax.lax.rsqrt(var + EPS)
    c = -m * rs
    t = x * rs + c
    out_ref[0] = t * gamma_ref[...] + beta_ref[...]


@jax.jit
def kernel(inputs_embeds, pos_table, ln_gamma, ln_beta):
    num_seq_blocks = SEQ_LEN // BLOCK_ROWS
    grid = (BATCH, num_seq_blocks)
    return pl.pallas_call(
        _ln_kernel,
        grid=grid,
        in_specs=[
            pl.BlockSpec((1, BLOCK_ROWS, D_MODEL), lambda j, i: (j, i, 0)),
            pl.BlockSpec((SEQ_LEN, D_MODEL), lambda j, i: (0, 0)),
            pl.BlockSpec((D_MODEL,), lambda j, i: (0,)),
            pl.BlockSpec((D_MODEL,), lambda j, i: (0,)),
        ],
        out_specs=pl.BlockSpec((1, BLOCK_ROWS, D_MODEL), lambda j, i: (j, i, 0)),
        out_shape=jax.ShapeDtypeStruct((BATCH, SEQ_LEN, D_MODEL), jnp.float32),
        compiler_params=pltpu.CompilerParams(
            dimension_semantics=("arbitrary", "arbitrary"),
        ),
    )(inputs_embeds, pos_table, ln_gamma, ln_beta)


# speedup vs baseline: 1.0731x; 1.0731x over previous
import jax
import jax.numpy as jnp
from jax.experimental import pallas as pl
from jax.experimental.pallas import tpu as pltpu

SEQ_LEN = 8192
D_MODEL = 768
BATCH = 4
EPS = 1e-12

BLOCK_ROWS = 2048


def _ln_kernel(x_ref, pos_ref, gamma_ref, beta_ref, out_ref):
    i = pl.program_id(1)
    x = x_ref[0] + pos_ref[pl.ds(i * BLOCK_ROWS, BLOCK_ROWS), :]
    inv_d = 1.0 / D_MODEL
    m = jnp.sum(x, axis=-1, keepdims=True) * inv_d
    m2 = jnp.sum(x * x, axis=-1, keepdims=True) * inv_d
    var = m2 - m * m
    rs = jax.lax.rsqrt(var + EPS)
    c = -m * rs
    t = x * rs + c
    out_ref[0] = t * gamma_ref[...] + beta_ref[...]


@jax.jit
def kernel(inputs_embeds, pos_table, ln_gamma, ln_beta):
    num_seq_blocks = SEQ_LEN // BLOCK_ROWS
    grid = (BATCH, num_seq_blocks)
    return pl.pallas_call(
        _ln_kernel,
        grid=grid,
        in_specs=[
            pl.BlockSpec((1, BLOCK_ROWS, D_MODEL), lambda j, i: (j, i, 0)),
            pl.BlockSpec((SEQ_LEN, D_MODEL), lambda j, i: (0, 0)),
            pl.BlockSpec((D_MODEL,), lambda j, i: (0,)),
            pl.BlockSpec((D_MODEL,), lambda j, i: (0,)),
        ],
        out_specs=pl.BlockSpec((1, BLOCK_ROWS, D_MODEL), lambda j, i: (j, i, 0)),
        out_shape=jax.ShapeDtypeStruct((BATCH, SEQ_LEN, D_MODEL), jnp.float32),
        compiler_params=pltpu.CompilerParams(
            dimension_semantics=("arbitrary", "arbitrary"),
        ),
    )(inputs_embeds, pos_table, ln_gamma, ln_beta)
